# baseline (speedup 1.0000x reference)
import jax
import jax.numpy as jnp
from jax.experimental import pallas as pl
from jax.experimental.pallas import tpu as pltpu

N_SUBJECTS = 16
TOK_BLK = 2048


def _fma_body(seg_ref, tp0_ref, tp1_ref, x_ref, o_ref):
    seg = seg_ref[0]
    te0 = jnp.zeros(seg.shape, jnp.float32)
    te1 = jnp.zeros(seg.shape, jnp.float32)
    for n in range(N_SUBJECTS):
        m = seg == n
        te0 = te0 + jnp.where(m, tp0_ref[n], 0.0)
        te1 = te1 + jnp.where(m, tp1_ref[n], 0.0)
    o_ref[...] = x_ref[...] * te1 + te0


def kernel(x, segment_ids, time_parameters0, time_parameters1):
    nb, tot = x.shape
    n_blocks = tot // TOK_BLK
    seg3 = segment_ids.reshape(n_blocks, 1, TOK_BLK).astype(jnp.int32)
    tp0 = time_parameters0.reshape(N_SUBJECTS)
    tp1 = time_parameters1.reshape(N_SUBJECTS)
    out = pl.pallas_call(
        _fma_body,
        grid=(n_blocks,),
        in_specs=[
            pl.BlockSpec((1, 1, TOK_BLK), lambda i: (i, 0, 0)),
            pl.BlockSpec(memory_space=pltpu.SMEM),
            pl.BlockSpec(memory_space=pltpu.SMEM),
            pl.BlockSpec((nb, TOK_BLK), lambda i: (0, i)),
        ],
        out_specs=pl.BlockSpec((nb, TOK_BLK), lambda i: (0, i)),
        out_shape=jax.ShapeDtypeStruct((nb, tot), jnp.float32),
    )(seg3, tp0, tp1, x)
    return out[:, :, None]


# speedup vs baseline: 1.0261x; 1.0261x over previous
import jax
import jax.numpy as jnp
from jax.experimental import pallas as pl
from jax.experimental.pallas import tpu as pltpu

N_SUBJECTS = 16
NB_BLK = 32


def _fma_body(seg_ref, tp0_ref, tp1_ref, x_ref, o_ref, te_ref):
    @pl.when(pl.program_id(0) == 0)
    def _expand():
        seg = seg_ref[0]
        te0 = jnp.zeros(seg.shape, jnp.float32)
        te1 = jnp.zeros(seg.shape, jnp.float32)
        for n in range(N_SUBJECTS):
            m = seg == n
            te0 = te0 + jnp.where(m, tp0_ref[n], 0.0)
            te1 = te1 + jnp.where(m, tp1_ref[n], 0.0)
        te_ref[0:1] = te0
        te_ref[1:2] = te1

    o_ref[...] = x_ref[...] * te_ref[1:2] + te_ref[0:1]


def kernel(x, segment_ids, time_parameters0, time_parameters1):
    nb, tot = x.shape
    n_blocks = nb // NB_BLK
    seg3 = segment_ids.reshape(1, 1, tot).astype(jnp.int32)
    tp0 = time_parameters0.reshape(N_SUBJECTS)
    tp1 = time_parameters1.reshape(N_SUBJECTS)
    out = pl.pallas_call(
        _fma_body,
        grid=(n_blocks,),
        in_specs=[
            pl.BlockSpec((1, 1, tot), lambda i: (0, 0, 0)),
            pl.BlockSpec(memory_space=pltpu.SMEM),
            pl.BlockSpec(memory_space=pltpu.SMEM),
            pl.BlockSpec((NB_BLK, tot), lambda i: (i, 0)),
        ],
        out_specs=pl.BlockSpec((NB_BLK, tot), lambda i: (i, 0)),
        out_shape=jax.ShapeDtypeStruct((nb, tot), jnp.float32),
        scratch_shapes=[pltpu.VMEM((2, tot), jnp.float32)],
    )(seg3, tp0, tp1, x)
    return out[:, :, None]


# speedup vs baseline: 2.6197x; 2.5532x over previous
import jax
import jax.numpy as jnp
from jax.experimental import pallas as pl
from jax.experimental.pallas import tpu as pltpu

N_SUBJECTS = 16
NB_BLK = 32


def _fma_body(seg_ref, tp0_ref, tp1_ref, x_ref, o_ref, te_ref):
    @pl.when(pl.program_id(0) == 0)
    def _expand():
        seg = seg_ref[0]
        te0 = jnp.zeros(seg.shape, jnp.float32)
        te1 = jnp.zeros(seg.shape, jnp.float32)
        for n in range(N_SUBJECTS):
            m = seg == n
            te0 = te0 + jnp.where(m, tp0_ref[n], 0.0)
            te1 = te1 + jnp.where(m, tp1_ref[n], 0.0)
        te_ref[0:1] = te0
        te_ref[1:2] = te1

    y = x_ref[...] * te_ref[1:2] + te_ref[0:1]
    o_ref[...] = y.reshape(o_ref.shape)


def kernel(x, segment_ids, time_parameters0, time_parameters1):
    nb, tot = x.shape
    n_blocks = nb // NB_BLK
    rows_per_blk = NB_BLK * tot // 1024
    seg3 = segment_ids.reshape(1, 1, tot).astype(jnp.int32)
    tp0 = time_parameters0.reshape(N_SUBJECTS)
    tp1 = time_parameters1.reshape(N_SUBJECTS)
    out = pl.pallas_call(
        _fma_body,
        grid=(n_blocks,),
        in_specs=[
            pl.BlockSpec((1, 1, tot), lambda i: (0, 0, 0)),
            pl.BlockSpec(memory_space=pltpu.SMEM),
            pl.BlockSpec(memory_space=pltpu.SMEM),
            pl.BlockSpec((NB_BLK, tot), lambda i: (i, 0)),
        ],
        out_specs=pl.BlockSpec((rows_per_blk, 8, 128), lambda i: (i, 0, 0)),
        out_shape=jax.ShapeDtypeStruct((nb * tot // 1024, 8, 128), jnp.float32),
        scratch_shapes=[pltpu.VMEM((2, tot), jnp.float32)],
    )(seg3, tp0, tp1, x)
    return out.reshape(nb, tot, 1)
